# baseline (speedup 1.0000x reference)
import jax
import jax.numpy as jnp
from jax.experimental import pallas as pl
from jax.experimental.pallas import tpu as pltpu

D_MODEL = 128
SEQ = 64
BATCH = 4096
SMALL = 64
BIG = 256
N_HEAD = 2
N_TAIL = 2
N_BIG = (BATCH - (N_HEAD + N_TAIL) * SMALL) // BIG
N_BIG_SLOTS = 4
LOOKAHEAD = 2

_SCHED = []
_off = 0
for _i in range(N_HEAD):
    _SCHED.append((_off, SMALL, False, _i))
    _off += SMALL
for _i in range(N_BIG):
    _SCHED.append((_off, BIG, True, _i % N_BIG_SLOTS))
    _off += BIG
for _i in range(N_TAIL):
    _SCHED.append((_off, SMALL, False, _i))
    _off += SMALL
assert _off == BATCH
_N = len(_SCHED)


def _body(inds_ref, x_hbm, pe_hbm, o_hbm, pe_v, fp, bigb, smallb,
          rsems, wsems, gsem):
    def bufslice(i):
        off, rows, is_big, slot = _SCHED[i]
        return (bigb if is_big else smallb).at[slot]

    def semidx(i):
        off, rows, is_big, slot = _SCHED[i]
        return slot if is_big else N_BIG_SLOTS + slot

    def read(i):
        off, rows, _, _ = _SCHED[i]
        return pltpu.make_async_copy(
            x_hbm.at[pl.ds(off, rows)], bufslice(i), rsems.at[semidx(i)])

    def write(i):
        off, rows, _, _ = _SCHED[i]
        return pltpu.make_async_copy(
            bufslice(i), o_hbm.at[pl.ds(off, rows)], wsems.at[semidx(i)])

    read(0).start()
    read(1).start()
    pltpu.make_async_copy(pe_hbm, pe_v, gsem).start()
    pltpu.make_async_copy(pe_hbm, pe_v, gsem).wait()

    def gather_row(j, _):
        idx = inds_ref[j]
        fp[pl.ds(j, 1), :] = pe_v[pl.ds(idx, 1), :]
        return 0

    jax.lax.fori_loop(0, SEQ, gather_row, 0)

    last_use = {(_SCHED[i][2], _SCHED[i][3]): i for i in range(LOOKAHEAD)}
    pending_writes = []
    for i in range(_N):
        _, rows, is_big, slot = _SCHED[i]
        ahead = i + LOOKAHEAD
        if ahead < _N:
            key = (_SCHED[ahead][2], _SCHED[ahead][3])
            prev = last_use.get(key)
            if prev is not None:
                write(prev).wait()
                pending_writes.remove(prev)
            read(ahead).start()
            last_use[key] = ahead

        read(i).wait()
        buf = bigb if is_big else smallb

        def add_rows(k, _, buf=buf, slot=slot):
            buf[slot, pl.ds(k * 8, 8)] = (
                buf[slot, pl.ds(k * 8, 8)] + fp[...][None, :, :])
            return 0

        jax.lax.fori_loop(0, rows // 8, add_rows, 0)
        write(i).start()
        pending_writes.append(i)

    for i in pending_writes:
        write(i).wait()


def kernel(x, x_node_inds, pe):
    inds = x_node_inds.astype(jnp.int32)
    pe64 = pe[:SEQ]

    grid_spec = pltpu.PrefetchScalarGridSpec(
        num_scalar_prefetch=1,
        grid=(1,),
        in_specs=[
            pl.BlockSpec(memory_space=pl.ANY),
            pl.BlockSpec(memory_space=pl.ANY),
        ],
        out_specs=pl.BlockSpec(memory_space=pl.ANY),
        scratch_shapes=[
            pltpu.VMEM((SEQ, D_MODEL), jnp.float32),
            pltpu.VMEM((SEQ, D_MODEL), jnp.float32),
            pltpu.VMEM((N_BIG_SLOTS, BIG, SEQ, D_MODEL), jnp.float32),
            pltpu.VMEM((N_HEAD, SMALL, SEQ, D_MODEL), jnp.float32),
            pltpu.SemaphoreType.DMA((N_BIG_SLOTS + N_HEAD,)),
            pltpu.SemaphoreType.DMA((N_BIG_SLOTS + N_HEAD,)),
            pltpu.SemaphoreType.DMA,
        ],
    )

    return pl.pallas_call(
        _body,
        grid_spec=grid_spec,
        out_shape=jax.ShapeDtypeStruct(x.shape, x.dtype),
        compiler_params=pltpu.CompilerParams(
            dimension_semantics=("arbitrary",),
        ),
    )(inds, x, pe64)


# speedup vs baseline: 1.0058x; 1.0058x over previous
import jax
import jax.numpy as jnp
from jax.experimental import pallas as pl
from jax.experimental.pallas import tpu as pltpu

D_MODEL = 128
SEQ = 64
BATCH_BLOCK = 256


def _body(inds_ref, x_ref, pe_ref, o_ref, fp_ref):
    @pl.when(pl.program_id(0) == 0)
    def _():
        def gather_row(j, _):
            idx = inds_ref[j]
            fp_ref[pl.ds(j, 1), :] = pe_ref[pl.ds(idx, 1), :]
            return 0

        jax.lax.fori_loop(0, SEQ, gather_row, 0)

    o_ref[...] = x_ref[...] + fp_ref[...][None, :, :]


def kernel(x, x_node_inds, pe):
    nb = x.shape[0] // BATCH_BLOCK
    inds = x_node_inds.astype(jnp.int32)
    pe64 = pe[:SEQ]

    grid_spec = pltpu.PrefetchScalarGridSpec(
        num_scalar_prefetch=1,
        grid=(nb,),
        in_specs=[
            pl.BlockSpec((BATCH_BLOCK, SEQ, D_MODEL), lambda i, inds_ref: (i, 0, 0)),
            pl.BlockSpec((SEQ, D_MODEL), lambda i, inds_ref: (0, 0)),
        ],
        out_specs=pl.BlockSpec((BATCH_BLOCK, SEQ, D_MODEL), lambda i, inds_ref: (i, 0, 0)),
        scratch_shapes=[pltpu.VMEM((SEQ, D_MODEL), jnp.float32)],
    )

    return pl.pallas_call(
        _body,
        grid_spec=grid_spec,
        out_shape=jax.ShapeDtypeStruct(x.shape, x.dtype),
        compiler_params=pltpu.CompilerParams(
            dimension_semantics=("arbitrary",),
        ),
    )(inds, x, pe64)
